# Initial kernel scaffold; baseline (speedup 1.0000x reference)
import jax
import jax.numpy as jnp
from jax.experimental import pallas as pl


def kernel(x, table):
    raise NotImplementedError("write your pallas kernel here")


# speedup vs baseline: 1.4797x; 1.4797x over previous
import functools

import jax
import jax.numpy as jnp
from jax import lax
from jax.experimental import pallas as pl
from jax.experimental.pallas import tpu as pltpu
from jax.experimental.pallas import tpu_sc as plsc


_IDX_W = 128
_K = 10
_G = _K * _IDX_W


@functools.lru_cache(maxsize=None)
def _make_gather(num_rows: int, feat: int, batch_flat: int):
    info = plsc.get_sparse_core_info()
    nc, ns = info.num_cores, info.num_subcores
    nw = nc * ns
    assert batch_flat % (nw * _G) == 0
    b_per_w = batch_flat // nw
    c_per_w = b_per_w // _G
    mesh = plsc.VectorSubcoreMesh(core_axis_name="core", subcore_axis_name="sub")

    @functools.partial(
        pl.kernel,
        out_type=jax.ShapeDtypeStruct((batch_flat, feat), jnp.float32),
        mesh=mesh,
        scratch_types=[
            pltpu.VMEM((_K, _IDX_W), jnp.int32),
            pltpu.VMEM((_G, feat), jnp.float32),
            pltpu.SemaphoreType.DMA,
        ],
        compiler_params=pltpu.CompilerParams(use_tc_tiling_on_sc=False),
    )
    def gather_kernel(idx_hbm, table_hbm, out_hbm, idx_v, rows_v, sem):
        w = lax.axis_index("sub") * nc + lax.axis_index("core")

        def chunk(ci, carry):
            pltpu.sync_copy(idx_hbm.at[w, ci], idx_v)
            copies = [
                pltpu.async_copy(
                    table_hbm.at[idx_v.at[j]],
                    rows_v.at[pl.ds(j * _IDX_W, _IDX_W)],
                    sem,
                )
                for j in range(_K)
            ]
            for cp in copies:
                cp.wait()
            pltpu.sync_copy(rows_v, out_hbm.at[pl.ds(w * b_per_w + ci * _G, _G)])
            return carry

        lax.fori_loop(0, c_per_w, chunk, 0)

    def run(idx_flat, table):
        idx4 = idx_flat.reshape(nw, c_per_w, _K, _IDX_W)
        return gather_kernel(idx4, table)

    return run


def kernel(x, table):
    b, h = x.shape[0], x.shape[1]
    run = _make_gather(table.shape[0], table.shape[1], b * h)
    out = run(x.reshape(-1), table)
    return out.reshape(b, h, table.shape[1])


# speedup vs baseline: 1.5035x; 1.0161x over previous
import functools

import jax
import jax.numpy as jnp
from jax import lax
from jax.experimental import pallas as pl
from jax.experimental.pallas import tpu as pltpu
from jax.experimental.pallas import tpu_sc as plsc


_IDX_W = 128
_K = 10
_G = _K * _IDX_W
_NBUF = 2


@functools.lru_cache(maxsize=None)
def _make_gather(num_rows: int, feat: int, batch_flat: int):
    info = plsc.get_sparse_core_info()
    nc, ns = info.num_cores, info.num_subcores
    nw = nc * ns
    assert batch_flat % (nw * _G * _NBUF) == 0
    b_per_w = batch_flat // nw
    n_chunks = b_per_w // _G
    n_streams = b_per_w // _IDX_W
    mesh = plsc.VectorSubcoreMesh(core_axis_name="core", subcore_axis_name="sub")

    @functools.partial(
        pl.kernel,
        out_type=jax.ShapeDtypeStruct((batch_flat, feat), jnp.float32),
        mesh=mesh,
        scratch_types=[
            pltpu.VMEM((n_streams, _IDX_W), jnp.int32),
            pltpu.VMEM((_NBUF, _G, feat), jnp.float32),
            pltpu.SemaphoreType.DMA,
            [pltpu.SemaphoreType.DMA] * _NBUF,
            [pltpu.SemaphoreType.DMA] * _NBUF,
        ],
        compiler_params=pltpu.CompilerParams(use_tc_tiling_on_sc=False),
    )
    def gather_kernel(idx_hbm, table_hbm, out_hbm, idx_v, rows_v, sem_i,
                      sems_g, sems_s):
        w = lax.axis_index("sub") * nc + lax.axis_index("core")
        pltpu.sync_copy(idx_hbm.at[w], idx_v)

        def fire(ci, b):
            return [
                pltpu.async_copy(
                    table_hbm.at[idx_v.at[ci * _K + j]],
                    rows_v.at[b, pl.ds(j * _IDX_W, _IDX_W)],
                    sems_g[b],
                )
                for j in range(_K)
            ]

        def store(ci, b):
            return pltpu.async_copy(
                rows_v.at[b],
                out_hbm.at[pl.ds(w * b_per_w + ci * _G, _G)],
                sems_s[b],
            )

        def body(c2, carry):
            ci0 = c2 * _NBUF
            gs = [fire(ci0 + b, b) for b in range(_NBUF)]
            stores = []
            for b in range(_NBUF):
                for cp in gs[b]:
                    cp.wait()
                stores.append(store(ci0 + b, b))
            for cp in stores:
                cp.wait()
            return carry

        lax.fori_loop(0, n_chunks // _NBUF, body, 0)

    def run(idx_flat, table):
        idx3 = idx_flat.reshape(nw, n_streams, _IDX_W)
        return gather_kernel(idx3, table)

    return run


def kernel(x, table):
    b, h = x.shape[0], x.shape[1]
    run = _make_gather(table.shape[0], table.shape[1], b * h)
    out = run(x.reshape(-1), table)
    return out.reshape(b, h, table.shape[1])
